# baseline (speedup 1.0000x reference)
import jax
import jax.numpy as jnp
from jax.experimental import pallas as pl
from jax.experimental.pallas import tpu as pltpu


def _body(x_ref, cw_ref, cb_ref, g_ref, b2_ref, o_ref):
    u = jnp.maximum(x_ref[...] * cw_ref[...] + cb_ref[...], 0.0)
    acc = jnp.dot(u, g_ref[...], preferred_element_type=jnp.float32)
    o_ref[...] = jnp.maximum(acc + b2_ref[...], 0.0)


def kernel(x, conv_w, conv_b, W0, b0, W1, b1, W2, b2):
    B, I = x.shape
    N = W2.shape[0]
    K = W2.shape[1] // 2
    G = jnp.zeros((I, N), x.dtype).at[33::2, :].set(W2[:, :K].T)
    BM = 1024
    out = pl.pallas_call(
        _body,
        grid=(B // BM,),
        in_specs=[
            pl.BlockSpec((BM, I), lambda i: (i, 0)),
            pl.BlockSpec((1, I), lambda i: (0, 0)),
            pl.BlockSpec((1, I), lambda i: (0, 0)),
            pl.BlockSpec((I, N), lambda i: (0, 0)),
            pl.BlockSpec((1, N), lambda i: (0, 0)),
        ],
        out_specs=pl.BlockSpec((BM, N), lambda i: (i, 0)),
        out_shape=jax.ShapeDtypeStruct((B, N), x.dtype),
        compiler_params=pltpu.CompilerParams(
            dimension_semantics=("parallel",),
        ),
    )(x, conv_w[None], conv_b[None], G, b2[None])
    return out
